# baseline (speedup 1.0000x reference)
import math

import jax
import jax.numpy as jnp
import numpy as np
from jax.experimental import pallas as pl
from jax.experimental.pallas import tpu as pltpu


def _ceil_to(v, m):
    return ((v + m - 1) // m) * m


def _fold_into(tri_ref, oh_ref, f_ref, w_ref, cw_ref):
    n = cw_ref.shape[0]
    nb = tri_ref.shape[0]
    nblk = (n + 254) // 128
    trib = tri_ref[...].astype(jnp.bfloat16)
    strips = []
    for j in range(nblk):
        strips.append(jax.lax.dot_general(
            trib[j:nb, :], oh_ref[:nb - j, :],
            (((0,), (0,)), ((), ())), preferred_element_type=jnp.float32))
    arr = jnp.concatenate(strips, axis=0)
    fb = f_ref[...]
    for k in range(7):
        sh = 1 << k
        rolled = jnp.roll(arr, -sh, axis=0)
        cond = jnp.broadcast_to(((fb >> k) & 1) == 1, arr.shape)
        arr = jnp.where(cond, rolled, arr)
    ut = arr[:n, :]
    c = jax.lax.broadcasted_iota(jnp.int32, (n, n), 0)
    r = jax.lax.broadcasted_iota(jnp.int32, (n, n), 1)
    ut = jnp.where(c >= r, ut, 0.0)
    ut = jnp.where(c == r, jnp.exp(ut), ut)
    acc = jax.lax.dot_general(
        ut.astype(jnp.bfloat16), w_ref[...].astype(jnp.bfloat16),
        (((1,), (1,)), ((), ())),
        preferred_element_type=jnp.float32)
    cw_ref[...] = acc.astype(cw_ref.dtype)


def _fused_kernel(tri_ref, oh_ref, f_ref, w_ref, x_ref, b_ref, o_ref, cw_sc):
    @pl.when(pl.program_id(1) == 0)
    def _():
        _fold_into(tri_ref, oh_ref, f_ref, w_ref, cw_sc)

    xb = x_ref[...].astype(jnp.bfloat16)
    acc = jnp.dot(xb, cw_sc[...], preferred_element_type=jnp.float32)
    o_ref[...] = (acc + b_ref[...]).astype(o_ref.dtype)


def kernel(x, weight, tri_vec, bias=None):
    out_features, in_features = weight.shape
    n = in_features
    dtype = x.dtype
    n_pad = _ceil_to(out_features, 128)

    rr = np.arange(n, dtype=np.int64)
    w = rr * n - (rr * (rr - 1)) // 2 - rr
    a, f = w // 128, w % 128
    tri_len = tri_vec.shape[0]
    nb = _ceil_to(tri_len, 128) // 128
    if nb * 128 != tri_len:
        tri_vec = jnp.concatenate(
            [tri_vec, jnp.zeros((nb * 128 - tri_len,), tri_vec.dtype)])
    tri2d = tri_vec.astype(jnp.float32).reshape(nb, 128)

    oh = np.zeros((nb, n), dtype=np.float32)
    oh[a, rr] = 1.0
    oh = jnp.asarray(oh.astype(jnp.bfloat16))
    fvec = jnp.asarray(f.astype(np.int32).reshape(1, n))

    if n_pad != out_features:
        weight = jnp.zeros((n_pad, n), weight.dtype).at[:out_features].set(weight)
    b = bias if bias is not None else jnp.zeros((out_features,), dtype)
    b = b.astype(jnp.float32)
    if n_pad != out_features:
        b2 = jnp.zeros((1, n_pad), jnp.float32).at[0, :out_features].set(b)
    else:
        b2 = b.reshape(1, n_pad)

    lead = x.shape[:-1]
    M = int(math.prod(lead)) if lead else 1
    x2d = x.reshape(M, n)

    tm = min(4096, _ceil_to(M, 8))
    grid_m = pl.cdiv(M, tm)
    cores = 2 if grid_m % 2 == 0 else 1
    gm2 = grid_m // cores

    out = pl.pallas_call(
        _fused_kernel,
        out_shape=jax.ShapeDtypeStruct((M, n_pad), jnp.float32),
        grid=(cores, gm2),
        in_specs=[
            pl.BlockSpec((nb, 128), lambda j, i: (0, 0),
                         pipeline_mode=pl.Buffered(1)),
            pl.BlockSpec((nb, n), lambda j, i: (0, 0),
                         pipeline_mode=pl.Buffered(1)),
            pl.BlockSpec((1, n), lambda j, i: (0, 0),
                         pipeline_mode=pl.Buffered(1)),
            pl.BlockSpec((n_pad, n), lambda j, i: (0, 0),
                         pipeline_mode=pl.Buffered(1)),
            pl.BlockSpec((tm, n), lambda j, i: (i * cores + j, 0)),
            pl.BlockSpec((1, n_pad), lambda j, i: (0, 0),
                         pipeline_mode=pl.Buffered(1)),
        ],
        out_specs=pl.BlockSpec((tm, n_pad), lambda j, i: (i * cores + j, 0)),
        scratch_shapes=[pltpu.VMEM((n, n_pad), jnp.bfloat16)],
        compiler_params=pltpu.CompilerParams(
            dimension_semantics=("parallel", "arbitrary"),
            vmem_limit_bytes=100 * 1024 * 1024,
        ),
    )(tri2d, oh, fvec, weight, x2d, b2)

    if n_pad != out_features:
        out = out[:, :out_features]
    return out.reshape(*lead, out_features).astype(dtype)


# speedup vs baseline: 1.0303x; 1.0303x over previous
import math

import jax
import jax.numpy as jnp
import numpy as np
from jax.experimental import pallas as pl
from jax.experimental.pallas import tpu as pltpu


def _ceil_to(v, m):
    return ((v + m - 1) // m) * m


def _fold_into(tri_ref, oh_ref, f_ref, w_ref, cw_ref):
    n = cw_ref.shape[0]
    nb = tri_ref.shape[0]
    nblk = (n + 254) // 128
    trib = tri_ref[...].astype(jnp.bfloat16)
    strips = []
    for j in range(nblk):
        strips.append(jax.lax.dot_general(
            trib[j:nb, :], oh_ref[:nb - j, :],
            (((0,), (0,)), ((), ())), preferred_element_type=jnp.float32))
    arr = jnp.concatenate(strips, axis=0)
    fb = f_ref[...]
    for k in range(7):
        sh = 1 << k
        rolled = jnp.roll(arr, -sh, axis=0)
        cond = jnp.broadcast_to(((fb >> k) & 1) == 1, arr.shape)
        arr = jnp.where(cond, rolled, arr)
    ut = arr[:n, :]
    c = jax.lax.broadcasted_iota(jnp.int32, (n, n), 0)
    r = jax.lax.broadcasted_iota(jnp.int32, (n, n), 1)
    ut = jnp.where(c >= r, ut, 0.0)
    ut = jnp.where(c == r, jnp.exp(ut), ut)
    acc = jax.lax.dot_general(
        ut.astype(jnp.bfloat16), w_ref[...].astype(jnp.bfloat16),
        (((1,), (1,)), ((), ())),
        preferred_element_type=jnp.float32)
    cw_ref[...] = acc.astype(cw_ref.dtype)


def _fused_kernel(tri_ref, oh_ref, f_ref, w_ref, x_ref, b_ref, o_ref, cw_sc):
    @pl.when(pl.program_id(1) == 0)
    def _():
        _fold_into(tri_ref, oh_ref, f_ref, w_ref, cw_sc)

    xb = x_ref[...].astype(jnp.bfloat16)
    acc = jnp.dot(xb, cw_sc[...], preferred_element_type=jnp.float32)
    o_ref[...] = (acc + b_ref[...]).astype(o_ref.dtype)


def kernel(x, weight, tri_vec, bias=None):
    out_features, in_features = weight.shape
    n = in_features
    dtype = x.dtype
    n_pad = _ceil_to(out_features, 128)

    rr = np.arange(n, dtype=np.int64)
    w = rr * n - (rr * (rr - 1)) // 2 - rr
    a, f = w // 128, w % 128
    tri_len = tri_vec.shape[0]
    nb = _ceil_to(tri_len, 128) // 128
    if nb * 128 != tri_len:
        tri_vec = jnp.concatenate(
            [tri_vec, jnp.zeros((nb * 128 - tri_len,), tri_vec.dtype)])
    tri2d = tri_vec.astype(jnp.float32).reshape(nb, 128)

    oh = np.zeros((nb, n), dtype=np.float32)
    oh[a, rr] = 1.0
    oh = jnp.asarray(oh.astype(jnp.bfloat16))
    fvec = jnp.asarray(f.astype(np.int32).reshape(1, n))

    if n_pad != out_features:
        weight = jnp.zeros((n_pad, n), weight.dtype).at[:out_features].set(weight)
    b = bias if bias is not None else jnp.zeros((out_features,), dtype)
    b = b.astype(jnp.float32)
    if n_pad != out_features:
        b2 = jnp.zeros((1, n_pad), jnp.float32).at[0, :out_features].set(b)
    else:
        b2 = b.reshape(1, n_pad)

    lead = x.shape[:-1]
    M = int(math.prod(lead)) if lead else 1
    x2d = x.reshape(M, n)

    tm = min(4096, _ceil_to(M, 8))
    grid_m = pl.cdiv(M, tm)
    cores = 2 if grid_m % 2 == 0 else 1
    gm2 = grid_m // cores

    out = pl.pallas_call(
        _fused_kernel,
        out_shape=jax.ShapeDtypeStruct((M, n_pad), jnp.float32),
        grid=(cores, gm2),
        in_specs=[
            pl.BlockSpec((nb, 128), lambda j, i: (0, 0),
                         pipeline_mode=pl.Buffered(1)),
            pl.BlockSpec((nb, n), lambda j, i: (0, 0),
                         pipeline_mode=pl.Buffered(1)),
            pl.BlockSpec((1, n), lambda j, i: (0, 0),
                         pipeline_mode=pl.Buffered(1)),
            pl.BlockSpec((n_pad, n), lambda j, i: (0, 0),
                         pipeline_mode=pl.Buffered(1)),
            pl.BlockSpec((tm, n), lambda j, i: (j * gm2 + i, 0)),
            pl.BlockSpec((1, n_pad), lambda j, i: (0, 0),
                         pipeline_mode=pl.Buffered(1)),
        ],
        out_specs=pl.BlockSpec((tm, n_pad), lambda j, i: (j * gm2 + i, 0)),
        scratch_shapes=[pltpu.VMEM((n, n_pad), jnp.bfloat16)],
        compiler_params=pltpu.CompilerParams(
            dimension_semantics=("parallel", "arbitrary"),
            vmem_limit_bytes=100 * 1024 * 1024,
        ),
    )(tri2d, oh, fvec, weight, x2d, b2)

    if n_pad != out_features:
        out = out[:, :out_features]
    return out.reshape(*lead, out_features).astype(dtype)
